# baseline (speedup 1.0000x reference)
import functools

import jax
import jax.numpy as jnp
from jax import lax
from jax.experimental import pallas as pl
from jax.experimental.pallas import tpu as pltpu
from jax.experimental.pallas import tpu_sc as plsc

BATCH = 4
SEQ = 2048
EMBED = 64
VOCAB = 100000
NCOL = (VOCAB + 127) // 128
PER_SC = 2 * SEQ


def _sc_dims():
    try:
        info = plsc.get_sparse_core_info()
        return info.num_cores, info.num_subcores
    except Exception:
        return 2, 16


@functools.cache
def _build():
    nc, ns = _sc_dims()
    mcol = -(-NCOL // ns)
    mesh = plsc.VectorSubcoreMesh(core_axis_name="c", subcore_axis_name="s")

    @functools.partial(
        pl.kernel,
        mesh=mesh,
        out_type=jax.ShapeDtypeStruct((BATCH, EMBED, SEQ), jnp.float32),
        scratch_types=[
            pltpu.VMEM((BATCH, SEQ), jnp.int32),
            pltpu.VMEM((4224,), jnp.int32),
            pltpu.VMEM((4224,), jnp.int32),
            pltpu.VMEM((4224,), jnp.int32),
            pltpu.VMEM((4224,), jnp.int32),
            pltpu.VMEM((EMBED, 128), jnp.float32),
            pltpu.VMEM((256, 128), jnp.float32),
            pltpu.VMEM((256,), jnp.int32),
            pltpu.VMEM((EMBED, 256), jnp.float32),
            pltpu.VMEM_SHARED((PER_SC + 128, 128), jnp.float32),
            pltpu.SemaphoreType.DMA,
        ],
        compiler_params=pltpu.CompilerParams(needs_layout_passes=False),
    )
    def emb(x_hbm, tokT_hbm, posT_hbm, outT_hbm,
            idxall, mpv, mpp, av, ap, tile, rowbuf, apf, posv,
            staging, sem):
        sc = lax.axis_index("c")
        k = lax.axis_index("s")
        i16 = lax.iota(jnp.int32, 16)

        def bc(s):
            return jnp.zeros((16,), jnp.int32) + s

        dump = bc(PER_SC + k)
        kv = bc(k)

        pltpu.sync_copy(x_hbm, idxall)

        def scan_chunk(q, cnt):
            bi = q // 128
            qq = q % 128
            v = idxall[2 * sc + bi, pl.ds(qq * 16, 16)]
            p = bc(bi * SEQ + qq * 16) + i16
            m = ((v >> 7) & bc(ns - 1)) == kv
            pref = plsc.cumsum(jnp.where(m, bc(1), bc(0)))
            slot = jnp.where(m, bc(cnt - 1) + pref, bc(4208))
            plsc.store_scatter(mpv, [slot], v)
            plsc.store_scatter(mpp, [slot], p)
            return cnt + pref[15]

        cnt = lax.fori_loop(0, 256, scan_chunk, jnp.int32(0))

        def reset_apf(q, carry):
            apf[pl.ds(q * 16, 16)] = dump
            return carry

        lax.fori_loop(0, 16, reset_apf, 0)

        nch = (cnt + 15) // 16

        def col_body(m, rb):
            c = jnp.minimum(k + ns * m, NCOL - 1)
            off = pl.multiple_of(c * 128, 128)
            pltpu.sync_copy(tokT_hbm.at[:, pl.ds(off, 128)], tile)

            def rescan(r, ac):
                vv = mpv[pl.ds(r * 16, 16)]
                pp = mpp[pl.ds(r * 16, 16)]
                mm = ((vv >> 7) == bc(c)) & ((bc(r * 16) + i16) < bc(cnt))
                pref = plsc.cumsum(jnp.where(mm, bc(1), bc(0)))
                slot = jnp.where(mm, bc(ac - 1) + pref, bc(4208))
                plsc.store_scatter(av, [slot], vv)
                plsc.store_scatter(ap, [slot], pp)
                return ac + pref[15]

            ac = lax.fori_loop(0, nch, rescan, jnp.int32(0))

            def extract(t, rb):
                full = rb + 16 > 256

                @pl.when(full)
                def _flush():
                    pltpu.sync_copy(rowbuf, staging.at[apf])
                    lax.fori_loop(0, 16, reset_apf, 0)

                rb = jnp.where(full, 0, rb)
                vlo = av[pl.ds(t * 16, 16)] & bc(127)
                pp = ap[pl.ds(t * 16, 16)]
                valid = (bc(t * 16) + i16) < bc(ac)
                apf[pl.ds(rb, 16)] = jnp.where(valid, pp, dump)
                slots = bc(rb) + i16
                for e in range(EMBED):
                    ecol = jnp.zeros((16,), jnp.int32) + e
                    vals = plsc.load_gather(tile, [ecol, vlo])
                    plsc.store_scatter(rowbuf, [slots, ecol], vals)
                return rb + 16

            return lax.fori_loop(0, (ac + 15) // 16, extract, rb)

        lax.fori_loop(0, mcol, col_body, jnp.int32(0))
        pltpu.sync_copy(rowbuf, staging.at[apf])
        plsc.subcore_barrier()

        bl = k // 8
        s0 = pl.multiple_of((k % 8) * 256, 128)
        base = pl.multiple_of(bl * SEQ + s0, 128)
        pltpu.sync_copy(staging.at[pl.ds(base, 256)], rowbuf)
        pltpu.sync_copy(posT_hbm.at[:, pl.ds(s0, 256)], posv)

        def trans(s16, carry):
            rows = bc(s16 * 16) + i16
            for e in range(EMBED):
                ecol = jnp.zeros((16,), jnp.int32) + e
                vals = plsc.load_gather(rowbuf, [rows, ecol])
                posv[e, pl.ds(s16 * 16, 16)] = (
                    vals + posv[e, pl.ds(s16 * 16, 16)]
                )
            return carry

        lax.fori_loop(0, 16, trans, 0)
        pltpu.sync_copy(posv, outT_hbm.at[2 * sc + bl, :, pl.ds(s0, 256)])

    return emb


def kernel(x, tok_table, pos_table):
    outT = _build()(x.astype(jnp.int32), tok_table.T, pos_table.T)
    return outT.transpose(0, 2, 1)


# speedup vs baseline: 1.4981x; 1.4981x over previous
import functools

import jax
import jax.numpy as jnp
from jax import lax
from jax.experimental import pallas as pl
from jax.experimental.pallas import tpu as pltpu
from jax.experimental.pallas import tpu_sc as plsc

BATCH = 4
SEQ = 2048
EMBED = 64
TOTAL = BATCH * SEQ


def _sc_dims():
    try:
        info = plsc.get_sparse_core_info()
        return info.num_cores, info.num_subcores
    except Exception:
        return 2, 16


@functools.cache
def _build():
    nc, ns = _sc_dims()
    nw = nc * ns
    bpw = TOTAL // nw
    assert TOTAL % nw == 0 and SEQ % bpw == 0
    mesh = plsc.VectorSubcoreMesh(core_axis_name="c", subcore_axis_name="s")

    @functools.partial(
        pl.kernel,
        mesh=mesh,
        out_type=jax.ShapeDtypeStruct((BATCH, SEQ, EMBED), jnp.float32),
        scratch_types=[
            pltpu.VMEM((bpw,), jnp.int32),
            pltpu.VMEM((bpw, EMBED), jnp.float32),
            pltpu.VMEM((bpw, EMBED), jnp.float32),
            pltpu.SemaphoreType.DMA,
        ],
        compiler_params=pltpu.CompilerParams(use_tc_tiling_on_sc=False),
    )
    def emb(x_hbm, tok_hbm, pos_hbm, out_hbm, idx_v, tok_v, pos_v, sem):
        wid = lax.axis_index("s") * nc + lax.axis_index("c")
        base = wid * bpw
        bidx = base // SEQ
        pos0 = base % SEQ
        pltpu.sync_copy(x_hbm.at[bidx, pl.ds(pos0, bpw)], idx_v)
        gather = pltpu.async_copy(tok_hbm.at[idx_v], tok_v, sem)
        pltpu.sync_copy(pos_hbm.at[pl.ds(pos0, bpw)], pos_v)
        gather.wait()

        def row_add(r, carry):
            for c in range(0, EMBED, 16):
                tok_v[r, pl.ds(c, 16)] = (
                    tok_v[r, pl.ds(c, 16)] + pos_v[r, pl.ds(c, 16)]
                )
            return carry

        lax.fori_loop(0, bpw, row_add, 0, unroll=4)
        pltpu.sync_copy(tok_v, out_hbm.at[bidx, pl.ds(pos0, bpw)])

    return emb


def kernel(x, tok_table, pos_table):
    return _build()(x.astype(jnp.int32), tok_table, pos_table)
